# baseline (speedup 1.0000x reference)
import functools

import jax
import jax.numpy as jnp
from jax import lax
from jax.experimental import pallas as pl
from jax.experimental.pallas import tpu as pltpu
from jax.experimental.pallas import tpu_sc as plsc

E = 320000
D = 128
A = 16
NC = 2
NS = 16
NW = NC * NS
NSPLIT = 2
EH = E // NSPLIT
EPW = EH // NW
CH = 40
NCHUNK = EPW // CH
DEPTH = 5
AHEAD = 3
NGRP = NCHUNK // DEPTH

_mesh = plsc.VectorSubcoreMesh(core_axis_name="c", subcore_axis_name="s")


@functools.partial(
    pl.kernel,
    out_type=jax.ShapeDtypeStruct((EH, 2 * D + A), jnp.float32),
    mesh=_mesh,
    scratch_types=[
        pltpu.VMEM((NCHUNK, 2 * CH), jnp.int32),
        [pltpu.VMEM((CH, D), jnp.float32)] * DEPTH,
        [pltpu.VMEM((CH, D), jnp.float32)] * DEPTH,
        [pltpu.VMEM((CH, A), jnp.float32)] * DEPTH,
        [pltpu.SemaphoreType.DMA] * DEPTH,
        [pltpu.SemaphoreType.DMA] * DEPTH,
    ],
)
def _edge_concat_half(x_hbm, ei_hbm, ea_hbm, out_hbm,
                      idx, sbufs, dbufs, abufs, rsems, wsems):
    wid = lax.axis_index("s") * NC + lax.axis_index("c")
    base = wid * EPW

    pltpu.sync_copy(ei_hbm.at[wid], idx)

    def issue_reads(j, s):
        gbase = base + j * CH
        pltpu.async_copy(x_hbm.at[idx.at[j, pl.ds(0, CH)]], sbufs[s], rsems[s])
        pltpu.async_copy(x_hbm.at[idx.at[j, pl.ds(CH, CH)]], dbufs[s], rsems[s])
        pltpu.async_copy(ea_hbm.at[pl.ds(gbase, CH)], abufs[s], rsems[s])

    def wait_reads(s):
        pltpu.make_async_copy(x_hbm.at[idx.at[0, pl.ds(0, CH)]], sbufs[s],
                              rsems[s]).wait()
        pltpu.make_async_copy(x_hbm.at[idx.at[0, pl.ds(CH, CH)]], dbufs[s],
                              rsems[s]).wait()
        pltpu.make_async_copy(ea_hbm.at[pl.ds(base, CH)], abufs[s],
                              rsems[s]).wait()

    def issue_writes(j, s):
        gbase = base + j * CH
        pltpu.async_copy(sbufs[s], out_hbm.at[pl.ds(gbase, CH), pl.ds(0, D)],
                         wsems[s])
        pltpu.async_copy(dbufs[s], out_hbm.at[pl.ds(gbase, CH), pl.ds(D, D)],
                         wsems[s])
        pltpu.async_copy(abufs[s],
                         out_hbm.at[pl.ds(gbase, CH), pl.ds(2 * D, A)],
                         wsems[s])

    def wait_writes(s):
        pltpu.make_async_copy(sbufs[s], out_hbm.at[pl.ds(base, CH), pl.ds(0, D)],
                              wsems[s]).wait()
        pltpu.make_async_copy(dbufs[s], out_hbm.at[pl.ds(base, CH), pl.ds(D, D)],
                              wsems[s]).wait()
        pltpu.make_async_copy(abufs[s],
                              out_hbm.at[pl.ds(base, CH), pl.ds(2 * D, A)],
                              wsems[s]).wait()

    for j in range(AHEAD):
        issue_reads(j, j)

    def grp(q, carry):
        j0 = DEPTH * q
        for b in range(DEPTH):
            j = j0 + b
            t = (b + AHEAD) % DEPTH

            @pl.when(j >= DEPTH - AHEAD)
            def _():
                wait_writes(t)

            @pl.when(j < NCHUNK - AHEAD)
            def _():
                issue_reads(j + AHEAD, t)

            wait_reads(b)
            issue_writes(j, b)
        return carry

    lax.fori_loop(0, NGRP, grp, 0)

    for b in range(DEPTH - AHEAD):
        wait_writes((NCHUNK - (DEPTH - AHEAD) + b) % DEPTH)


def kernel(x, edge_index, edge_attr):
    ei = edge_index.astype(jnp.int32).reshape(2, NSPLIT, NW, NCHUNK, CH)
    ea = edge_attr.reshape(NSPLIT, EH, A)
    halves = []
    for k in range(NSPLIT):
        eik = jnp.concatenate([ei[0, k], ei[1, k]], axis=-1)
        halves.append(_edge_concat_half(x, eik, ea[k]))
    return jnp.concatenate(halves, axis=0)


# speedup vs baseline: 1.3613x; 1.3613x over previous
import functools

import jax
import jax.numpy as jnp
from jax import lax
from jax.experimental import pallas as pl
from jax.experimental.pallas import tpu as pltpu
from jax.experimental.pallas import tpu_sc as plsc

E = 320000
D = 128
A = 16
NC = 2
NS = 16
NW = NC * NS
EPW = E // NW
CH = 80
NCHUNK = EPW // CH
NPAIR = NCHUNK // 2

_mesh = plsc.VectorSubcoreMesh(core_axis_name="c", subcore_axis_name="s")


@functools.partial(
    pl.kernel,
    out_type=jax.ShapeDtypeStruct((E, 2 * D + A), jnp.float32),
    mesh=_mesh,
    scratch_types=[
        pltpu.VMEM((NCHUNK, CH), jnp.int32),
        pltpu.VMEM((NCHUNK, CH), jnp.int32),
        pltpu.VMEM((CH, D), jnp.float32),
        pltpu.VMEM((CH, D), jnp.float32),
        pltpu.VMEM((CH, D), jnp.float32),
        pltpu.VMEM((CH, D), jnp.float32),
        pltpu.VMEM((CH, A), jnp.float32),
        pltpu.VMEM((CH, A), jnp.float32),
        pltpu.SemaphoreType.DMA,
        pltpu.SemaphoreType.DMA,
        pltpu.SemaphoreType.DMA,
        pltpu.SemaphoreType.DMA,
    ],
)
def _edge_concat(x_hbm, ei_hbm, ea_hbm, out_hbm,
                 sidx, didx, sbuf0, sbuf1, dbuf0, dbuf1, abuf0, abuf1,
                 rsem0, rsem1, wsem0, wsem1):
    wid = lax.axis_index("s") * NC + lax.axis_index("c")
    base = wid * EPW

    pltpu.sync_copy(ei_hbm.at[0, wid], sidx)
    pltpu.sync_copy(ei_hbm.at[1, wid], didx)

    def issue_reads(j, sbuf, dbuf, abuf, rsem):
        gbase = base + j * CH
        pltpu.async_copy(x_hbm.at[sidx.at[j]], sbuf, rsem)
        pltpu.async_copy(x_hbm.at[didx.at[j]], dbuf, rsem)
        pltpu.async_copy(ea_hbm.at[pl.ds(gbase, CH)], abuf, rsem)

    def wait_reads(sbuf, dbuf, abuf, rsem):
        pltpu.make_async_copy(x_hbm.at[sidx.at[0]], sbuf, rsem).wait()
        pltpu.make_async_copy(x_hbm.at[didx.at[0]], dbuf, rsem).wait()
        pltpu.make_async_copy(ea_hbm.at[pl.ds(base, CH)], abuf, rsem).wait()

    def issue_writes(j, sbuf, dbuf, abuf, wsem):
        gbase = base + j * CH
        pltpu.async_copy(sbuf, out_hbm.at[pl.ds(gbase, CH), pl.ds(0, D)], wsem)
        pltpu.async_copy(dbuf, out_hbm.at[pl.ds(gbase, CH), pl.ds(D, D)], wsem)
        pltpu.async_copy(abuf, out_hbm.at[pl.ds(gbase, CH), pl.ds(2 * D, A)], wsem)

    def wait_writes(sbuf, dbuf, abuf, wsem):
        pltpu.make_async_copy(sbuf, out_hbm.at[pl.ds(base, CH), pl.ds(0, D)], wsem).wait()
        pltpu.make_async_copy(dbuf, out_hbm.at[pl.ds(base, CH), pl.ds(D, D)], wsem).wait()
        pltpu.make_async_copy(abuf, out_hbm.at[pl.ds(base, CH), pl.ds(2 * D, A)], wsem).wait()

    issue_reads(0, sbuf0, dbuf0, abuf0, rsem0)

    def pair(g, carry):
        j0 = 2 * g
        j1 = j0 + 1

        @pl.when(g > 0)
        def _():
            wait_writes(sbuf1, dbuf1, abuf1, wsem1)
        issue_reads(j1, sbuf1, dbuf1, abuf1, rsem1)
        wait_reads(sbuf0, dbuf0, abuf0, rsem0)
        issue_writes(j0, sbuf0, dbuf0, abuf0, wsem0)

        wait_writes(sbuf0, dbuf0, abuf0, wsem0)
        issue_reads(j0 + 2, sbuf0, dbuf0, abuf0, rsem0)
        wait_reads(sbuf1, dbuf1, abuf1, rsem1)
        issue_writes(j1, sbuf1, dbuf1, abuf1, wsem1)
        return carry

    lax.fori_loop(0, NPAIR, pair, 0)

    wait_writes(sbuf1, dbuf1, abuf1, wsem1)
    wait_reads(sbuf0, dbuf0, abuf0, rsem0)
    issue_writes(NCHUNK - 1, sbuf0, dbuf0, abuf0, wsem0)
    wait_writes(sbuf0, dbuf0, abuf0, wsem0)


def kernel(x, edge_index, edge_attr):
    ei = edge_index.astype(jnp.int32).reshape(2, NW, NCHUNK, CH)
    return _edge_concat(x, ei, edge_attr)
